# baseline (speedup 1.0000x reference)
import functools

import jax
import jax.numpy as jnp
from jax import lax
from jax.experimental import pallas as pl
from jax.experimental.pallas import tpu as pltpu
from jax.experimental.pallas import tpu_sc as plsc

_BATCH = 16384
_HIST = 50
_D = 32

_NC = 2
_NS = 16
_NW = _NC * _NS
_BPW = _BATCH // _NW
_L = 16

_mesh = plsc.VectorSubcoreMesh(core_axis_name="c", subcore_axis_name="s")


@functools.partial(
    pl.kernel,
    out_type=jax.ShapeDtypeStruct((_HIST, _D, _BATCH), jnp.float32),
    mesh=_mesh,
    scratch_types=[
        pltpu.VMEM((_HIST, _BPW), jnp.int32),
        pltpu.VMEM((2, _BPW, _D), jnp.float32),
        pltpu.VMEM((2, _D, _BPW), jnp.float32),
        pltpu.SemaphoreType.DMA((2,)),
        pltpu.SemaphoreType.DMA((2,)),
    ],
    compiler_params=pltpu.CompilerParams(
        use_tc_tiling_on_sc=False, needs_layout_passes=False
    ),
)
def _sc_gather(idxt_hbm, table_hbm, out_hbm, idx_v, rows_v, t_v, gsem, ssem):
    wid = lax.axis_index("s") * _NC + lax.axis_index("c")
    b0 = wid * _BPW
    pltpu.sync_copy(idxt_hbm.at[:, pl.ds(b0, _BPW)], idx_v)

    def start_gather(h, p):
        for c in range(4):
            pltpu.make_async_copy(
                table_hbm.at[idx_v.at[h, pl.ds(c * 128, 128)]],
                rows_v.at[p, pl.ds(c * 128, 128)],
                gsem.at[p],
            ).start()

    def wait_gather(h, p):
        pltpu.make_async_copy(
            table_hbm.at[pl.ds(0, _BPW)], rows_v.at[p], gsem.at[p]
        ).wait()

    def store_desc(h, p):
        return pltpu.make_async_copy(
            t_v.at[p],
            out_hbm.at[h, :, pl.ds(b0, _BPW)],
            ssem.at[p],
        )

    def transpose(p):
        def q_body(q, carry):
            bvec = q * _L + lax.iota(jnp.int32, _L)
            for d in range(_D):
                vals = plsc.load_gather(
                    rows_v.at[p], [bvec, jnp.full((_L,), d, jnp.int32)]
                )
                t_v[p, d, pl.ds(q * _L, _L)] = vals
            return carry

        lax.fori_loop(0, _BPW // _L, q_body, 0)

    start_gather(0, 0)

    def outer(i, carry):
        for par in range(2):
            h = 2 * i + par

            @pl.when(h + 1 < _HIST)
            def _(h=h, par=par):
                start_gather(h + 1, 1 - par)

            wait_gather(h, par)

            @pl.when(h >= 2)
            def _(h=h, par=par):
                store_desc(h - 2, par).wait()

            transpose(par)
            store_desc(h, par).start()
        return carry

    lax.fori_loop(0, _HIST // 2, outer, 0)

    store_desc(_HIST - 2, 0).wait()
    store_desc(_HIST - 1, 1).wait()


def kernel(inputs, table):
    out_phys = _sc_gather(inputs.T, table)
    return out_phys.transpose(2, 0, 1)


# speedup vs baseline: 1.2944x; 1.2944x over previous
import functools

import jax
import jax.numpy as jnp
from jax import lax
from jax.experimental import pallas as pl
from jax.experimental.pallas import tpu as pltpu
from jax.experimental.pallas import tpu_sc as plsc

_BATCH = 16384
_HIST = 50
_D = 32

_NC = 2
_NS = 16
_NW = _NC * _NS
_BPW = _BATCH // _NW
_NBUF = 5
_N_OUTER = _HIST // _NBUF

_mesh = plsc.VectorSubcoreMesh(core_axis_name="c", subcore_axis_name="s")


@functools.partial(
    pl.kernel,
    out_type=jax.ShapeDtypeStruct((_HIST, _BATCH, _D), jnp.float32),
    mesh=_mesh,
    scratch_types=[
        pltpu.VMEM((_HIST, _BPW), jnp.int32),
        pltpu.VMEM((_NBUF, _BPW, _D), jnp.float32),
        pltpu.SemaphoreType.DMA((_NBUF,)),
        pltpu.SemaphoreType.DMA((_NBUF,)),
    ],
    compiler_params=pltpu.CompilerParams(use_tc_tiling_on_sc=False),
)
def _sc_gather(idxt_hbm, table_hbm, out_hbm, idx_v, rows_v, gsem, ssem):
    wid = lax.axis_index("s") * _NC + lax.axis_index("c")
    b0 = wid * _BPW
    pltpu.sync_copy(idxt_hbm.at[:, pl.ds(b0, _BPW)], idx_v)

    def start_gather(h, b):
        for c in range(4):
            pltpu.make_async_copy(
                table_hbm.at[idx_v.at[h, pl.ds(c * 128, 128)]],
                rows_v.at[b, pl.ds(c * 128, 128)],
                gsem.at[b],
            ).start()

    def wait_gather(h, b):
        pltpu.make_async_copy(
            out_hbm.at[h, pl.ds(b0, _BPW)], rows_v.at[b], gsem.at[b]
        ).wait()

    def store_desc(h, b):
        return pltpu.make_async_copy(
            rows_v.at[b],
            out_hbm.at[h, pl.ds(b0, _BPW)],
            ssem.at[b],
        )

    for b in range(_NBUF - 1):
        start_gather(b, b)

    def outer(i, carry):
        t0 = i * _NBUF
        for b in range(_NBUF):
            t = t0 + b
            h_next = t + _NBUF - 1
            bn = (b - 1) % _NBUF

            def refill(t=t, h_next=h_next, bn=bn, guard_prev=(b == 0)):
                if guard_prev:
                    @pl.when(t >= 1)
                    def _():
                        store_desc(t - 1, bn).wait()
                else:
                    store_desc(t - 1, bn).wait()
                start_gather(h_next, bn)

            pl.when(h_next < _HIST)(refill)

            wait_gather(t, b)
            store_desc(t, b).start()
        return carry

    lax.fori_loop(0, _N_OUTER, outer, 0)

    for b in range(_NBUF):
        store_desc(_HIST - _NBUF + b, b).wait()


def kernel(inputs, table):
    return _sc_gather(inputs.T, table).transpose(1, 0, 2)
